# baseline (speedup 1.0000x reference)
import jax
import jax.numpy as jnp
from jax import lax
from jax.experimental import pallas as pl
from jax.experimental.pallas import tpu as pltpu
from jax.experimental.pallas import tpu_sc as plsc

N_NODES = 10000
N_EDGES = 320000
D = 128
L = 16
NC = 2
NS = 16
NW = NC * NS
E_PER_W = N_EDGES // NW
C = 80
N_CHUNKS = E_PER_W // C
NV = D // L
K = 4


def _sc_body(x_hbm, src_hbm, dst_hbm, dist_hbm, dir_hbm,
             src_idx0, src_idx1, src_idx2, src_idx3,
             dst_idx0, dst_idx1, dst_idx2, dst_idx3,
             src_rows0, src_rows1, src_rows2, src_rows3,
             dst_rows0, dst_rows1, dst_rows2, dst_rows3,
             dir_v0, dir_v1, dir_v2, dir_v3,
             dist_v0, dist_v1, dist_v2, dist_v3,
             is0, is1, is2, is3, id0, id1, id2, id3,
             gs0, gs1, gs2, gs3, gd0, gd1, gd2, gd3,
             oa0, oa1, oa2, oa3, ob0, ob1, ob2, ob3):
    src_idx = (src_idx0, src_idx1, src_idx2, src_idx3)
    dst_idx = (dst_idx0, dst_idx1, dst_idx2, dst_idx3)
    src_rows = (src_rows0, src_rows1, src_rows2, src_rows3)
    dst_rows = (dst_rows0, dst_rows1, dst_rows2, dst_rows3)
    dir_v = (dir_v0, dir_v1, dir_v2, dir_v3)
    dist_v = (dist_v0, dist_v1, dist_v2, dist_v3)
    sem_is = (is0, is1, is2, is3)
    sem_id = (id0, id1, id2, id3)
    sem_gs = (gs0, gs1, gs2, gs3)
    sem_gd = (gd0, gd1, gd2, gd3)
    sem_oa = (oa0, oa1, oa2, oa3)
    sem_ob = (ob0, ob1, ob2, ob3)

    wid = lax.axis_index("s") * NC + lax.axis_index("c")
    base_w = wid * E_PER_W
    lane = lax.iota(jnp.int32, L)

    def ebase(c):
        return base_w + c * C

    def start_idx(c, b):
        pltpu.make_async_copy(
            src_hbm.at[pl.ds(ebase(c), C)], src_idx[b], sem_is[b]).start()
        pltpu.make_async_copy(
            dst_hbm.at[pl.ds(ebase(c), C)], dst_idx[b], sem_id[b]).start()

    def wait_idx(b):
        pltpu.make_async_copy(
            src_hbm.at[pl.ds(0, C)], src_idx[b], sem_is[b]).wait()
        pltpu.make_async_copy(
            dst_hbm.at[pl.ds(0, C)], dst_idx[b], sem_id[b]).wait()

    def start_gather(b):
        pltpu.make_async_copy(
            x_hbm.at[src_idx[b]], src_rows[b], sem_gs[b]).start()
        pltpu.make_async_copy(
            x_hbm.at[dst_idx[b]], dst_rows[b], sem_gd[b]).start()

    def wait_gather(b):
        pltpu.make_async_copy(
            x_hbm.at[src_idx[b]], src_rows[b], sem_gs[b]).wait()
        pltpu.make_async_copy(
            x_hbm.at[dst_idx[b]], dst_rows[b], sem_gd[b]).wait()

    def start_out(c, b):
        pltpu.make_async_copy(
            dir_v[b], dir_hbm.at[pl.ds(ebase(c), C)], sem_oa[b]).start()
        pltpu.make_async_copy(
            dist_v[b], dist_hbm.at[pl.ds(ebase(c), C)], sem_ob[b]).start()

    def wait_out(b):
        pltpu.make_async_copy(
            dir_v[b], dir_hbm.at[pl.ds(0, C)], sem_oa[b]).wait()
        pltpu.make_async_copy(
            dist_v[b], dist_hbm.at[pl.ds(0, C)], sem_ob[b]).wait()

    shuf = lambda v, perm: jnp.take_along_axis(
        v, perm, axis=0, mode="promise_in_bounds")
    perms = [lane ^ jnp.int32(1 << t) for t in range(4)]
    dmask = lane < 1
    dzero = lane * 0

    def compute(b):
        sr, dr, dv = src_rows[b], dst_rows[b], dir_v[b]

        @plsc.parallel_loop(0, C, 1, unroll=1)
        def edge_group(e):
            du = []
            p = None
            for v in range(NV):
                d = sr[e, pl.ds(v * L, L)] - dr[e, pl.ds(v * L, L)]
                du.append(d)
                p = d * d if p is None else p + d * d
            for t in range(4):
                p = p + shuf(p, perms[t])
            m = jnp.maximum(p, jnp.float32(1e-30))
            i = lax.bitcast_convert_type(m, jnp.int32)
            i = jnp.int32(0x5F3759DF) - lax.shift_right_arithmetic(i, 1)
            y = lax.bitcast_convert_type(i, jnp.float32)
            hm = jnp.float32(0.5) * m
            for _ in range(2):
                t = y * y
                t = hm * t
                y = y * (jnp.float32(1.5) - t)
            dist = m * y
            a = jnp.float32(1.0) + dist
            i = lax.bitcast_convert_type(a, jnp.int32)
            i = jnp.int32(0x7EF311C3) - i
            z = lax.bitcast_convert_type(i, jnp.float32)
            for _ in range(3):
                z = z * (jnp.float32(2.0) - a * z)
            plsc.store_scatter(dist_v[b], [e + dzero], dist, mask=dmask)
            for v in range(NV):
                dv[e, pl.ds(v * L, L)] = du[v] * z

    for b in range(K):
        start_idx(b, b)
    for b in range(K - 1):
        wait_idx(b)
        start_gather(b)

    def quad(j, carry):
        for b in range(K):
            c = K * j + b
            b3 = (b + 3) % K
            @pl.when(c < N_CHUNKS - (K - 1))
            def _():
                wait_idx(b3)
                start_gather(b3)
            wait_gather(b)
            @pl.when(c < N_CHUNKS - K)
            def _():
                start_idx(c + K, b)
            @pl.when(c >= K)
            def _():
                wait_out(b)
            compute(b)
            start_out(c, b)
        return carry

    lax.fori_loop(0, (N_CHUNKS - 1) // K, quad, 0)

    c = N_CHUNKS - 1
    b = c % K
    wait_gather(b)
    wait_out(b)
    compute(b)
    start_out(c, b)
    wait_out(1)
    wait_out(2)
    wait_out(3)
    wait_out(0)


@jax.jit
def _add_edges_sc(x, src, dst):
    mesh = plsc.VectorSubcoreMesh(core_axis_name="c", subcore_axis_name="s")
    fn = pl.kernel(
        _sc_body,
        mesh=mesh,
        compiler_params=pltpu.CompilerParams(needs_layout_passes=False),
        out_type=[
            jax.ShapeDtypeStruct((N_EDGES,), jnp.float32),
            jax.ShapeDtypeStruct((N_EDGES, D), jnp.float32),
        ],
        scratch_types=(
            [pltpu.VMEM((C,), jnp.int32)] * 8
            + [pltpu.VMEM((C, D), jnp.float32)] * 12
            + [pltpu.VMEM((C,), jnp.float32)] * 4
            + [pltpu.SemaphoreType.DMA] * 24
        ),
    )
    return fn(x, src, dst)


def kernel(x, edge_index):
    src = edge_index[0].astype(jnp.int32)
    dst = edge_index[1].astype(jnp.int32)
    dist, direction = _add_edges_sc(x, src, dst)
    return dist, direction


# speedup vs baseline: 1.0169x; 1.0169x over previous
import jax
import jax.numpy as jnp
from jax import lax
from jax.experimental import pallas as pl
from jax.experimental.pallas import tpu as pltpu
from jax.experimental.pallas import tpu_sc as plsc

N_NODES = 10000
N_EDGES = 320000
D = 128
L = 16
NC = 2
NS = 16
NW = NC * NS
E_PER_W = N_EDGES // NW
C = 80
N_CHUNKS = E_PER_W // C
NV = D // L
K = 3


def _sc_body(x_hbm, src_hbm, dst_hbm, dist_hbm, dir_hbm,
             src_all, dst_all, dist_all,
             src_rows0, src_rows1, src_rows2,
             dst_rows0, dst_rows1, dst_rows2,
             dir_v0, dir_v1, dir_v2,
             si, di,
             gs0, gs1, gs2, gd0, gd1, gd2, oa0, oa1, oa2, od):
    src_rows = (src_rows0, src_rows1, src_rows2)
    dst_rows = (dst_rows0, dst_rows1, dst_rows2)
    dir_v = (dir_v0, dir_v1, dir_v2)
    sem_gs = (gs0, gs1, gs2)
    sem_gd = (gd0, gd1, gd2)
    sem_oa = (oa0, oa1, oa2)

    wid = lax.axis_index("s") * NC + lax.axis_index("c")
    base_w = wid * E_PER_W
    lane = lax.iota(jnp.int32, L)

    def ebase(c):
        return base_w + c * C

    def start_gather(c, b):
        pltpu.make_async_copy(
            x_hbm.at[src_all.at[pl.ds(c * C, C)]], src_rows[b],
            sem_gs[b]).start()
        pltpu.make_async_copy(
            x_hbm.at[dst_all.at[pl.ds(c * C, C)]], dst_rows[b],
            sem_gd[b]).start()

    def wait_gather(b):
        pltpu.make_async_copy(
            x_hbm.at[src_all.at[pl.ds(0, C)]], src_rows[b], sem_gs[b]).wait()
        pltpu.make_async_copy(
            x_hbm.at[dst_all.at[pl.ds(0, C)]], dst_rows[b], sem_gd[b]).wait()

    def start_out(c, b):
        pltpu.make_async_copy(
            dir_v[b], dir_hbm.at[pl.ds(ebase(c), C)], sem_oa[b]).start()

    def wait_out(b):
        pltpu.make_async_copy(
            dir_v[b], dir_hbm.at[pl.ds(0, C)], sem_oa[b]).wait()

    shuf = lambda v, perm: jnp.take_along_axis(
        v, perm, axis=0, mode="promise_in_bounds")
    perms = [lane ^ jnp.int32(1 << t) for t in range(4)]
    dmask = lane < 1
    dzero = lane * 0

    def compute(c, b):
        sr, dr, dv = src_rows[b], dst_rows[b], dir_v[b]
        cbase = c * C

        @plsc.parallel_loop(0, C, 1, unroll=1)
        def edge_group(e):
            du = []
            p = None
            for v in range(NV):
                d = sr[e, pl.ds(v * L, L)] - dr[e, pl.ds(v * L, L)]
                du.append(d)
                p = d * d if p is None else p + d * d
            for t in range(4):
                p = p + shuf(p, perms[t])
            m = jnp.maximum(p, jnp.float32(1e-30))
            i = lax.bitcast_convert_type(m, jnp.int32)
            i = jnp.int32(0x5F3759DF) - lax.shift_right_arithmetic(i, 1)
            y = lax.bitcast_convert_type(i, jnp.float32)
            hm = jnp.float32(0.5) * m
            for _ in range(2):
                t = y * y
                t = hm * t
                y = y * (jnp.float32(1.5) - t)
            dist = m * y
            a = jnp.float32(1.0) + dist
            i = lax.bitcast_convert_type(a, jnp.int32)
            i = jnp.int32(0x7EF311C3) - i
            z = lax.bitcast_convert_type(i, jnp.float32)
            for _ in range(3):
                z = z * (jnp.float32(2.0) - a * z)
            plsc.store_scatter(dist_all, [cbase + e + dzero], dist, mask=dmask)
            for v in range(NV):
                dv[e, pl.ds(v * L, L)] = du[v] * z

    pltpu.make_async_copy(
        src_hbm.at[pl.ds(base_w, E_PER_W)], src_all, si).start()
    pltpu.make_async_copy(
        dst_hbm.at[pl.ds(base_w, E_PER_W)], dst_all, di).start()
    pltpu.make_async_copy(
        src_hbm.at[pl.ds(base_w, E_PER_W)], src_all, si).wait()
    pltpu.make_async_copy(
        dst_hbm.at[pl.ds(base_w, E_PER_W)], dst_all, di).wait()
    start_gather(0, 0)
    start_gather(1, 1)

    def triple(j, carry):
        for b in range(K):
            c = K * j + b
            b2 = (b + 2) % K
            @pl.when(c < N_CHUNKS - 2)
            def _():
                start_gather(c + 2, b2)
            wait_gather(b)
            @pl.when(c >= K)
            def _():
                wait_out(b)
            compute(c, b)
            start_out(c, b)
        return carry

    lax.fori_loop(0, (N_CHUNKS - 2) // K, triple, 0)

    for c in (N_CHUNKS - 2, N_CHUNKS - 1):
        b = c % K
        wait_gather(b)
        wait_out(b)
        compute(c, b)
        start_out(c, b)
    wait_out(2)
    wait_out(0)
    wait_out(1)
    pltpu.make_async_copy(
        dist_all, dist_hbm.at[pl.ds(base_w, E_PER_W)], od).start()
    pltpu.make_async_copy(
        dist_all, dist_hbm.at[pl.ds(base_w, E_PER_W)], od).wait()


@jax.jit
def _add_edges_sc(x, src, dst):
    mesh = plsc.VectorSubcoreMesh(core_axis_name="c", subcore_axis_name="s")
    fn = pl.kernel(
        _sc_body,
        mesh=mesh,
        compiler_params=pltpu.CompilerParams(needs_layout_passes=False),
        out_type=[
            jax.ShapeDtypeStruct((N_EDGES,), jnp.float32),
            jax.ShapeDtypeStruct((N_EDGES, D), jnp.float32),
        ],
        scratch_types=(
            [pltpu.VMEM((E_PER_W,), jnp.int32)] * 2
            + [pltpu.VMEM((E_PER_W,), jnp.float32)]
            + [pltpu.VMEM((C, D), jnp.float32)] * 9
            + [pltpu.SemaphoreType.DMA] * 12
        ),
    )
    return fn(x, src, dst)


def kernel(x, edge_index):
    src = edge_index[0].astype(jnp.int32)
    dst = edge_index[1].astype(jnp.int32)
    dist, direction = _add_edges_sc(x, src, dst)
    return dist, direction
